# baseline (speedup 1.0000x reference)
import functools

import jax
import jax.numpy as jnp
from jax import lax
from jax.experimental import pallas as pl
from jax.experimental.pallas import tpu as pltpu
from jax.experimental.pallas import tpu_sc as plsc

_NUM_EXPERTS = 64
_DROP_RATE = 0.1
_NC = 2
_NS = 16
_L = 16

_B = 4
_S = 8192
_K = 8
_TOTAL = _B * _S * _K
_PER_W = _TOTAL // (_NC * _NS)
_GROUPS = _PER_W // (_K * 128)
_ITERS = _GROUPS * (128 // _L)

_MASK_W0 = 0x77EFDFFF
_MASK_W1 = 0xFDEFFFAF


def _sc_body(w_hbm, idx_hbm, out_hbm, w_v, idx_v, out_v, sem_w, sem_i):
    wid = lax.axis_index("s") * _NC + lax.axis_index("c")
    base = wid * _PER_W
    cp_w = pltpu.async_copy(w_hbm.at[pl.ds(base, _PER_W)], w_v, sem_w)
    cp_i = pltpu.async_copy(idx_hbm.at[pl.ds(base, _PER_W)], idx_v, sem_i)
    cp_w.wait()
    cp_i.wait()

    w0 = jnp.full((_L,), _MASK_W0, dtype=jnp.uint32)
    w1 = jnp.full((_L,), _MASK_W1, dtype=jnp.uint32)

    @plsc.parallel_loop(0, _ITERS, unroll=8)
    def _(i):
        off0 = (i >> 3) * (_K * 128) + (i & 7) * _L
        s_regs = []
        denom = None
        for k in range(_K):
            off = off0 + k * 128
            idx = idx_v[pl.ds(off, _L)]
            sh = (idx & 31).astype(jnp.uint32)
            bits = jnp.where(idx < 32, w0 >> sh, w1 >> sh) & 1
            s_k = w_v[pl.ds(off, _L)] * bits.astype(jnp.float32)
            s_regs.append(s_k)
            denom = s_k if denom is None else denom + s_k
        r = 1.0 / (denom + 1e-10)
        for k in range(_K):
            out_v[pl.ds(off0 + k * 128, _L)] = s_regs[k] * r

    pltpu.sync_copy(out_v, out_hbm.at[pl.ds(base, _PER_W)])


@functools.partial(
    pl.kernel,
    out_type=jax.ShapeDtypeStruct((_TOTAL,), jnp.float32),
    mesh=plsc.VectorSubcoreMesh(
        core_axis_name="c", subcore_axis_name="s",
        num_cores=_NC, num_subcores=_NS),
    scratch_types=[
        pltpu.VMEM((_PER_W,), jnp.float32),
        pltpu.VMEM((_PER_W,), jnp.int32),
        pltpu.VMEM((_PER_W,), jnp.float32),
        pltpu.SemaphoreType.DMA,
        pltpu.SemaphoreType.DMA,
    ],
    compiler_params=pltpu.CompilerParams(needs_layout_passes=False),
    name="expert_dropout_sc",
)
def _expert_dropout_sc(w_hbm, idx_hbm, out_hbm, w_v, idx_v, out_v,
                       sem_w, sem_i):
    _sc_body(w_hbm, idx_hbm, out_hbm, w_v, idx_v, out_v, sem_w, sem_i)


def _phys_flat(x):
    return (x.transpose(0, 2, 1).reshape(_B, _K, _S // 128, 128)
            .transpose(0, 2, 1, 3).reshape(_TOTAL))


def _unphys(flat):
    return (flat.reshape(_B, _S // 128, _K, 128).transpose(0, 2, 1, 3)
            .reshape(_B, _K, _S).transpose(0, 2, 1))


def kernel(expert_weights, expert_indices):
    idx = expert_indices
    if idx.dtype != jnp.int32:
        idx = idx.astype(jnp.int32)
    out = _expert_dropout_sc(_phys_flat(expert_weights), _phys_flat(idx))
    return (_unphys(out), expert_indices)
